# Initial kernel scaffold; baseline (speedup 1.0000x reference)
import jax
import jax.numpy as jnp
from jax.experimental import pallas as pl


def kernel(news, table):
    raise NotImplementedError("write your pallas kernel here")


# speedup vs baseline: 14.4967x; 14.4967x over previous
import functools

import jax
import jax.numpy as jnp
from jax import lax
from jax.experimental import pallas as pl
from jax.experimental.pallas import tpu as pltpu
from jax.experimental.pallas import tpu_sc as plsc

NC = 2
NS = 16
NW = NC * NS
LANES = 16


@functools.partial(jax.jit, static_argnames=("batch", "length", "lpad", "dim"))
def _embed_mean(idx_t, table, *, batch, length, lpad, dim):
    bpw = batch // NW
    mesh = plsc.VectorSubcoreMesh(core_axis_name="c", subcore_axis_name="s")

    @functools.partial(
        pl.kernel,
        out_type=jax.ShapeDtypeStruct((batch, dim), jnp.float32),
        mesh=mesh,
        scratch_types=[
            pltpu.VMEM((lpad, bpw), jnp.int32),
            pltpu.VMEM((bpw, dim), jnp.float32),
            pltpu.VMEM((1, dim), jnp.float32),
            pltpu.VMEM((bpw,), jnp.float32),
            pltpu.SemaphoreType.DMA,
        ],
    )
    def body(idx_hbm, table_hbm, out_hbm, idx_v, acc, t0, cnt_v, sem):
        wid = lax.axis_index("s") * NC + lax.axis_index("c")
        pltpu.sync_copy(idx_hbm.at[wid], idx_v)
        pltpu.sync_copy(table_hbm.at[pl.ds(0, 1)], t0)

        zero = jnp.zeros((LANES,), jnp.float32)

        def zero_body(b, _):
            for k in range(dim // LANES):
                acc[b, pl.ds(k * LANES, LANES)] = zero
            return ()

        lax.fori_loop(0, bpw, zero_body, ())

        def fire(j, _):
            pltpu.async_copy(table_hbm.at[idx_v.at[j]], acc, sem, add=True)
            return ()

        lax.fori_loop(0, length, fire, ())

        def cnt_body(j, carry):
            accs = list(carry)
            for k in range(bpw // LANES):
                v = idx_v[j, pl.ds(k * LANES, LANES)]
                accs[k] = accs[k] + jnp.where(v == 0, 1.0, 0.0).astype(jnp.float32)
            return tuple(accs)

        counts = lax.fori_loop(
            0, length, cnt_body,
            tuple(jnp.zeros((LANES,), jnp.float32) for _ in range(bpw // LANES)),
        )
        for k in range(bpw // LANES):
            cnt_v[pl.ds(k * LANES, LANES)] = counts[k]

        def drain(j, _):
            pltpu.make_async_copy(table_hbm.at[idx_v.at[0]], acc, sem).wait()
            return ()

        lax.fori_loop(0, length, drain, ())

        scale = jnp.float32(1.0 / length)

        def fix_body(b, _):
            base = (b // LANES) * LANES
            lane = b - base
            cv = cnt_v[pl.ds(base, LANES)]
            dnums = lax.GatherDimensionNumbers(
                offset_dims=(), collapsed_slice_dims=(0,), start_index_map=(0,))
            cb = lax.gather(
                cv, jnp.full((LANES, 1), lane, jnp.int32), dnums, (1,),
                mode=lax.GatherScatterMode.PROMISE_IN_BOUNDS)
            for k in range(dim // LANES):
                a = acc[b, pl.ds(k * LANES, LANES)]
                t = t0[0, pl.ds(k * LANES, LANES)]
                acc[b, pl.ds(k * LANES, LANES)] = (a - cb * t) * scale
            return ()

        lax.fori_loop(0, bpw, fix_body, ())
        pltpu.sync_copy(acc, out_hbm.at[pl.ds(wid * bpw, bpw)])

    return body(idx_t, table)


def kernel(news, table):
    batch, seq = news.shape
    _, dim = table.shape
    length = seq // 4
    lpad = -(-length // 8) * 8
    bpw = batch // NW
    idx_t = (
        news[:, :length]
        .reshape(NW, bpw, length)
        .transpose(0, 2, 1)
    )
    idx_t = jnp.pad(idx_t, ((0, 0), (0, lpad - length), (0, 0)))
    return _embed_mean(idx_t, table, batch=batch, length=length, lpad=lpad, dim=dim)


# speedup vs baseline: 16.0411x; 1.1065x over previous
import functools

import jax
import jax.numpy as jnp
from jax import lax
from jax.experimental import pallas as pl
from jax.experimental.pallas import tpu as pltpu
from jax.experimental.pallas import tpu_sc as plsc

NC = 2
NS = 16
NW = NC * NS
LANES = 16


@functools.partial(jax.jit, static_argnames=("batch", "length", "lpad", "dim"))
def _embed_mean(idx_t, table, *, batch, length, lpad, dim):
    bpw = batch // NW
    mesh = plsc.VectorSubcoreMesh(core_axis_name="c", subcore_axis_name="s")

    @functools.partial(
        pl.kernel,
        out_type=jax.ShapeDtypeStruct((batch, dim), jnp.float32),
        mesh=mesh,
        scratch_types=[
            pltpu.VMEM((lpad, bpw), jnp.int32),
            pltpu.VMEM((bpw, dim), jnp.float32),
            pltpu.VMEM((1, dim), jnp.float32),
            pltpu.VMEM((bpw,), jnp.float32),
            pltpu.SemaphoreType.DMA,
            pltpu.SemaphoreType.DMA,
        ],
    )
    def body(idx_hbm, table_hbm, out_hbm, idx_v, acc, t0, cnt_v, sem, lsem):
        wid = lax.axis_index("s") * NC + lax.axis_index("c")
        idx_cp = pltpu.async_copy(idx_hbm.at[wid], idx_v, lsem)
        t0_cp = pltpu.async_copy(table_hbm.at[pl.ds(0, 1)], t0, lsem)

        zero = jnp.zeros((LANES,), jnp.float32)

        def zero_body(b, _):
            for k in range(dim // LANES):
                acc[b, pl.ds(k * LANES, LANES)] = zero
            return ()

        lax.fori_loop(0, bpw, zero_body, ())
        idx_cp.wait()
        t0_cp.wait()

        def fire(j, _):
            pltpu.async_copy(table_hbm.at[idx_v.at[j]], acc, sem, add=True)
            return ()

        lax.fori_loop(0, length, fire, ())

        def cnt_body(j, carry):
            accs = list(carry)
            for k in range(bpw // LANES):
                v = idx_v[j, pl.ds(k * LANES, LANES)]
                accs[k] = accs[k] + jnp.where(v == 0, 1.0, 0.0).astype(jnp.float32)
            return tuple(accs)

        counts = lax.fori_loop(
            0, length, cnt_body,
            tuple(jnp.zeros((LANES,), jnp.float32) for _ in range(bpw // LANES)),
        )
        for k in range(bpw // LANES):
            cnt_v[pl.ds(k * LANES, LANES)] = counts[k]

        def drain(j, _):
            pltpu.make_async_copy(table_hbm.at[idx_v.at[0]], acc, sem).wait()
            return ()

        lax.fori_loop(0, length, drain, ())

        scale = jnp.float32(1.0 / length)
        ts = [t0[0, pl.ds(k * LANES, LANES)] * scale for k in range(dim // LANES)]
        dnums = lax.GatherDimensionNumbers(
            offset_dims=(), collapsed_slice_dims=(0,), start_index_map=(0,))

        def fix_body(b, _):
            base = (b // LANES) * LANES
            lane = b - base
            cv = cnt_v[pl.ds(base, LANES)]
            cb = lax.gather(
                cv, jnp.full((LANES, 1), lane, jnp.int32), dnums, (1,),
                mode=lax.GatherScatterMode.PROMISE_IN_BOUNDS)
            for k in range(dim // LANES):
                a = acc[b, pl.ds(k * LANES, LANES)]
                acc[b, pl.ds(k * LANES, LANES)] = a * scale - cb * ts[k]
            return ()

        lax.fori_loop(0, bpw, fix_body, ())
        pltpu.sync_copy(acc, out_hbm.at[pl.ds(wid * bpw, bpw)])

    return body(idx_t, table)


def kernel(news, table):
    batch, seq = news.shape
    _, dim = table.shape
    length = seq // 4
    lpad = -(-length // 8) * 8
    bpw = batch // NW
    idx_t = (
        news[:, :lpad]
        .reshape(NW, bpw, lpad)
        .transpose(0, 2, 1)
    )
    return _embed_mean(idx_t, table, batch=batch, length=length, lpad=lpad, dim=dim)
